# Initial kernel scaffold; baseline (speedup 1.0000x reference)
import jax
import jax.numpy as jnp
from jax.experimental import pallas as pl


def kernel(x, conv_w, conv_b, bn_gamma, bn_beta, proj_w, proj_b):
    raise NotImplementedError("write your pallas kernel here")


# speedup vs baseline: 1.9502x; 1.9502x over previous
import jax
import jax.numpy as jnp
from jax.experimental import pallas as pl

_B, _N, _D, _K, _O = 16, 512, 3, 20, 512
_NEG = -3.0e38
_PREC_HI = jax.lax.Precision.HIGHEST
_PREC_MID = jax.lax.Precision.HIGHEST


def _stage_a(x_ref, xT_ref, wcomb_ref, ac_ref, idx_ref, stats_ref):
    b = pl.program_id(0)
    xb = x_ref[0]
    xT = xT_ref[0]
    ac = jax.lax.dot_general(xb, wcomb_ref[...], (((1,), (0,)), ((), ())),
                             precision=_PREC_HI)
    ac_ref[0] = ac
    A = ac[:, :_O]
    C = ac[:, _O:]
    inner = jax.lax.dot_general(xb, xT, (((1,), (0,)), ((), ())),
                                precision=jax.lax.Precision.DEFAULT)
    xxrow = jnp.sum(xT * xT, axis=0, keepdims=True)
    score = 2.0 * inner - xxrow
    miota = jax.lax.broadcasted_iota(jnp.int32, (_N, _N), 1)
    sel = jnp.zeros((_N, _N), jnp.float32)
    idx_cols = []
    for _ in range(_K):
        rowmax = jnp.max(score, axis=1, keepdims=True)
        cand = jnp.where(score == rowmax, miota, _N)
        midx = jnp.min(cand, axis=1, keepdims=True)
        onehot = miota == midx
        sel = sel + onehot.astype(jnp.float32)
        score = jnp.where(onehot, _NEG, score)
        idx_cols.append(midx)
    idx_cols.append(jnp.zeros((_N, 128 - _K), jnp.int32))
    idx_ref[0] = jnp.concatenate(idx_cols, axis=1)
    S = jax.lax.dot_general(sel, C, (((1,), (0,)), ((), ())),
                            precision=_PREC_HI)
    cnt = jnp.sum(sel, axis=0, keepdims=True)
    s4 = jax.lax.dot_general(cnt, C * C, (((1,), (0,)), ((), ())),
                             precision=_PREC_HI)
    s1 = jnp.sum(A, axis=0, keepdims=True)
    s2 = jnp.sum(A * A, axis=0, keepdims=True)
    s3 = jnp.sum(S, axis=0, keepdims=True)
    s5 = jnp.sum(A * S, axis=0, keepdims=True)
    part = jnp.concatenate(
        [s1, s2, s3, s4, s5, jnp.zeros((3, _O), jnp.float32)], axis=0)

    @pl.when(b == 0)
    def _():
        stats_ref[...] = part

    @pl.when(b > 0)
    def _():
        stats_ref[...] = stats_ref[...] + part


def _stage_b(ac_ref, idx_ref, stats_ref, params_ref, pwT_ref, out_ref):
    ac = ac_ref[0]
    A = ac[:, :_O]
    C = ac[:, _O:]
    st = stats_ref[...]
    gamma = params_ref[0:1, :]
    beta = params_ref[1:2, :]
    pb = params_ref[3:4, :]
    bnk = float(_B * _N * _K)
    s1, s2, s3, s4, s5 = (st[i:i + 1, :] for i in range(5))
    mean0 = (_K * s1 + s3) / bnk
    e2 = (_K * s2 + 2.0 * s5 + s4) / bnk
    var = e2 - mean0 * mean0
    scale = gamma * jax.lax.rsqrt(var + 1e-5)
    Ap = scale * (A - mean0) + beta
    Cp = scale * C
    miota = jax.lax.broadcasted_iota(jnp.int32, (_N, _N), 1)
    idx = idx_ref[0]
    acc = jnp.full((_N, 1), _NEG, jnp.float32)
    for k in range(_K):
        onehot = (miota == idx[:, k:k + 1]).astype(jnp.float32)
        G = jax.lax.dot_general(onehot, Cp, (((1,), (0,)), ((), ())),
                                precision=_PREC_MID)
        acc = jnp.maximum(acc, jnp.max(Ap + G, axis=1, keepdims=True))
    v = jnp.maximum(acc, 0.0)
    eye = (jax.lax.broadcasted_iota(jnp.int32, (_N, _N), 0) == miota)
    vrow = jnp.sum(eye.astype(jnp.float32) * v, axis=0, keepdims=True)
    out = jax.lax.dot_general(vrow, pwT_ref[...], (((1,), (0,)), ((), ())),
                              precision=_PREC_HI) + pb
    out_ref[0] = out


def kernel(x, conv_w, conv_b, bn_gamma, bn_beta, proj_w, proj_b):
    f32 = jnp.float32
    x = x.astype(f32)
    w1 = conv_w[:, :_D]
    w2 = conv_w[:, _D:]
    wcomb = jnp.zeros((128, 2 * _O), f32)
    wcomb = wcomb.at[:_D, :_O].set((w1 - w2).T).at[:_D, _O:].set(w2.T)
    xpad = jnp.pad(x, ((0, 0), (0, 0), (0, 128 - _D)))
    xT = jnp.swapaxes(xpad, 1, 2)
    params = jnp.zeros((8, _O), f32)
    params = (params.at[0].set(bn_gamma).at[1].set(bn_beta)
              .at[2].set(conv_b).at[3].set(proj_b))
    pwT = proj_w.T

    ac, idx, stats = pl.pallas_call(
        _stage_a,
        grid=(_B,),
        in_specs=[
            pl.BlockSpec((1, _N, 128), lambda b: (b, 0, 0)),
            pl.BlockSpec((1, 128, _N), lambda b: (b, 0, 0)),
            pl.BlockSpec((128, 2 * _O), lambda b: (0, 0)),
        ],
        out_specs=[
            pl.BlockSpec((1, _N, 2 * _O), lambda b: (b, 0, 0)),
            pl.BlockSpec((1, _N, 128), lambda b: (b, 0, 0)),
            pl.BlockSpec((8, _O), lambda b: (0, 0)),
        ],
        out_shape=[
            jax.ShapeDtypeStruct((_B, _N, 2 * _O), f32),
            jax.ShapeDtypeStruct((_B, _N, 128), jnp.int32),
            jax.ShapeDtypeStruct((8, _O), f32),
        ],
    )(xpad, xT, wcomb)

    out = pl.pallas_call(
        _stage_b,
        grid=(_B,),
        in_specs=[
            pl.BlockSpec((1, _N, 2 * _O), lambda b: (b, 0, 0)),
            pl.BlockSpec((1, _N, 128), lambda b: (b, 0, 0)),
            pl.BlockSpec((8, _O), lambda b: (0, 0)),
            pl.BlockSpec((8, _O), lambda b: (0, 0)),
            pl.BlockSpec((_N, _O), lambda b: (0, 0)),
        ],
        out_specs=pl.BlockSpec((1, 1, _O), lambda b: (b, 0, 0)),
        out_shape=jax.ShapeDtypeStruct((_B, 1, _O), f32),
    )(ac, idx, stats, params, pwT)
    return out.reshape(_B, _O)


# speedup vs baseline: 4.7802x; 2.4511x over previous
import jax
import jax.numpy as jnp
from jax.experimental import pallas as pl

_B, _N, _D, _K, _O = 16, 512, 3, 20, 512
_NEG = -3.0e38
_PREC_HI = jax.lax.Precision.HIGHEST
_PREC_MID = jax.lax.Precision.HIGHEST


def _stage_a(x_ref, xT_ref, wcomb_ref, ac_ref, idx_ref, stats_ref):
    b = pl.program_id(0)
    xb = x_ref[0]
    xT = xT_ref[0]
    ac = jax.lax.dot_general(xb, wcomb_ref[...], (((1,), (0,)), ((), ())),
                             precision=_PREC_HI)
    ac_ref[0] = ac
    A = ac[:, :_O]
    C = ac[:, _O:]
    inner = jax.lax.dot_general(xb, xT, (((1,), (0,)), ((), ())),
                                precision=jax.lax.Precision.DEFAULT)
    xxrow = jnp.sum(xT * xT, axis=0, keepdims=True)
    score = 2.0 * inner - xxrow
    miota = jax.lax.broadcasted_iota(jnp.int32, (_N, _N), 1)
    sel = jnp.zeros((_N, _N), jnp.float32)
    idx_cols = []
    for _ in range(_K):
        rowmax = jnp.max(score, axis=1, keepdims=True)
        cand = jnp.where(score == rowmax, miota, _N)
        midx = jnp.min(cand, axis=1, keepdims=True)
        onehot = miota == midx
        sel = sel + onehot.astype(jnp.float32)
        score = jnp.where(onehot, _NEG, score)
        idx_cols.append(midx)
    idx_cols.append(jnp.zeros((_N, 128 - _K), jnp.int32))
    idx_ref[0] = jnp.concatenate(idx_cols, axis=1)
    S = jax.lax.dot_general(sel, C, (((1,), (0,)), ((), ())),
                            precision=_PREC_HI)
    cnt = jnp.sum(sel, axis=0, keepdims=True)
    s4 = jax.lax.dot_general(cnt, C * C, (((1,), (0,)), ((), ())),
                             precision=_PREC_HI)
    s1 = jnp.sum(A, axis=0, keepdims=True)
    s2 = jnp.sum(A * A, axis=0, keepdims=True)
    s3 = jnp.sum(S, axis=0, keepdims=True)
    s5 = jnp.sum(A * S, axis=0, keepdims=True)
    part = jnp.concatenate(
        [s1, s2, s3, s4, s5, jnp.zeros((3, _O), jnp.float32)], axis=0)

    @pl.when(b == 0)
    def _():
        stats_ref[...] = part

    @pl.when(b > 0)
    def _():
        stats_ref[...] = stats_ref[...] + part


def _stage_b(ac_ref, idx_ref, stats_ref, params_ref, pwT_ref, out_ref):
    ac = ac_ref[0]
    A = ac[:, :_O]
    C = ac[:, _O:]
    st = stats_ref[...]
    gamma = params_ref[0:1, :]
    beta = params_ref[1:2, :]
    pb = params_ref[3:4, :]
    bnk = float(_B * _N * _K)
    s1, s2, s3, s4, s5 = (st[i:i + 1, :] for i in range(5))
    mean0 = (_K * s1 + s3) / bnk
    e2 = (_K * s2 + 2.0 * s5 + s4) / bnk
    var = e2 - mean0 * mean0
    scale = gamma * jax.lax.rsqrt(var + 1e-5)
    Ap = scale * (A - mean0) + beta
    Cp = scale * C
    miota = jax.lax.broadcasted_iota(jnp.int32, (_N, _N), 1)
    idx = idx_ref[0]
    acc = jnp.full((_N, 1), _NEG, jnp.float32)
    for k in range(_K):
        onehot = (miota == idx[:, k:k + 1]).astype(jnp.float32)
        G = jax.lax.dot_general(onehot, Cp, (((1,), (0,)), ((), ())),
                                precision=jax.lax.Precision.DEFAULT)
        acc = jnp.maximum(acc, jnp.max(Ap + G, axis=1, keepdims=True))
    v = jnp.maximum(acc, 0.0)
    eye = (jax.lax.broadcasted_iota(jnp.int32, (_N, _N), 0) == miota)
    vrow = jnp.sum(eye.astype(jnp.float32) * v, axis=0, keepdims=True)
    out = jax.lax.dot_general(vrow, pwT_ref[...], (((1,), (0,)), ((), ())),
                              precision=_PREC_HI) + pb
    out_ref[0] = out


def kernel(x, conv_w, conv_b, bn_gamma, bn_beta, proj_w, proj_b):
    f32 = jnp.float32
    x = x.astype(f32)
    w1 = conv_w[:, :_D]
    w2 = conv_w[:, _D:]
    wcomb = jnp.zeros((128, 2 * _O), f32)
    wcomb = wcomb.at[:_D, :_O].set((w1 - w2).T).at[:_D, _O:].set(w2.T)
    xpad = jnp.pad(x, ((0, 0), (0, 0), (0, 128 - _D)))
    xT = jnp.swapaxes(xpad, 1, 2)
    params = jnp.zeros((8, _O), f32)
    params = (params.at[0].set(bn_gamma).at[1].set(bn_beta)
              .at[2].set(conv_b).at[3].set(proj_b))
    pwT = proj_w.T

    ac, idx, stats = pl.pallas_call(
        _stage_a,
        grid=(_B,),
        in_specs=[
            pl.BlockSpec((1, _N, 128), lambda b: (b, 0, 0)),
            pl.BlockSpec((1, 128, _N), lambda b: (b, 0, 0)),
            pl.BlockSpec((128, 2 * _O), lambda b: (0, 0)),
        ],
        out_specs=[
            pl.BlockSpec((1, _N, 2 * _O), lambda b: (b, 0, 0)),
            pl.BlockSpec((1, _N, 128), lambda b: (b, 0, 0)),
            pl.BlockSpec((8, _O), lambda b: (0, 0)),
        ],
        out_shape=[
            jax.ShapeDtypeStruct((_B, _N, 2 * _O), f32),
            jax.ShapeDtypeStruct((_B, _N, 128), jnp.int32),
            jax.ShapeDtypeStruct((8, _O), f32),
        ],
    )(xpad, xT, wcomb)

    out = pl.pallas_call(
        _stage_b,
        grid=(_B,),
        in_specs=[
            pl.BlockSpec((1, _N, 2 * _O), lambda b: (b, 0, 0)),
            pl.BlockSpec((1, _N, 128), lambda b: (b, 0, 0)),
            pl.BlockSpec((8, _O), lambda b: (0, 0)),
            pl.BlockSpec((8, _O), lambda b: (0, 0)),
            pl.BlockSpec((_N, _O), lambda b: (0, 0)),
        ],
        out_specs=pl.BlockSpec((1, 1, _O), lambda b: (b, 0, 0)),
        out_shape=jax.ShapeDtypeStruct((_B, 1, _O), f32),
    )(ac, idx, stats, params, pwT)
    return out.reshape(_B, _O)
